# baseline (speedup 1.0000x reference)
import functools

import jax
import jax.numpy as jnp
from jax.experimental import pallas as pl
from jax.experimental.pallas import tpu as pltpu

C = 256
N_HIDDEN = 5
BE = 1280


def _leaky(x):
    return jnp.where(x > 0, x, 0.01 * x)


def _dot(a, b):
    return jax.lax.dot_general(a, b, (((1,), (0,)), ((), ())),
                               preferred_element_type=jnp.float32)


def _mlp_tail(g, w_ref, b_ref):
    h = _leaky(g)
    for i in range(N_HIDDEN - 1):
        h = _leaky(_dot(h, w_ref[i]) + b_ref[i : i + 1, :])
    return _dot(h, w_ref[N_HIDDEN - 1]) + b_ref[N_HIDDEN - 1 : N_HIDDEN, :]


def _mlp5_kernel(g_ref, w_ref, b_ref, out_ref):
    out_ref[...] = _mlp_tail(g_ref[...], w_ref, b_ref)


def _mlp5_pair_kernel(glo_ref, ghi_ref, ew_ref, w_ref, b_ref, out_ref):
    m = _mlp_tail(glo_ref[...], w_ref, b_ref) + _mlp_tail(ghi_ref[...], w_ref, b_ref)
    out_ref[...] = m * ew_ref[...]


def _e2_head_kernel(glo_ref, ghi_ref, ef1g_ref, eff_ref, ew_ref,
                    wm_ref, bm_ref, we_ref, be_ref, wh_ref, bh_ref,
                    out_ref):
    ew = ew_ref[...]
    m = _mlp_tail(glo_ref[...], wm_ref, bm_ref) + _mlp_tail(ghi_ref[...], wm_ref, bm_ref)
    ef2pre = m * ew
    z = _dot(ef2pre, we_ref[0]) + ef1g_ref[...] + be_ref[0:1, :]
    h = _leaky(z)
    for i in range(1, N_HIDDEN):
        h = _leaky(_dot(h, we_ref[i]) + be_ref[i : i + 1, :])
    ef2 = _leaky(_dot(h, we_ref[N_HIDDEN]) + be_ref[N_HIDDEN : N_HIDDEN + 1, :])
    e1 = _dot(ef2, wh_ref[...]) + eff_ref[...] + bh_ref[0:1, :]
    s0 = jnp.sum(e1 * bh_ref[1:2, :], axis=1, keepdims=True) + bh_ref[3:4, 0:1]
    s1 = jnp.sum(e1 * bh_ref[2:3, :], axis=1, keepdims=True) + bh_ref[3:4, 1:2]
    s0 = jax.nn.sigmoid(s0)
    s1 = jax.nn.sigmoid(s1)
    mx = jnp.maximum(s0, s1)
    z0 = jnp.exp(s0 - mx)
    z1 = jnp.exp(s1 - mx)
    tot = z0 + z1
    out_ref[...] = jnp.concatenate([z0 / tot, z1 / tot], axis=1)


def _full_spec(shape):
    return pl.BlockSpec(shape, lambda i: tuple(0 for _ in shape))


def _mlp5(g, wh, bh):
    n = g.shape[0]
    grid = (n // BE,)
    return pl.pallas_call(
        _mlp5_kernel,
        grid=grid,
        in_specs=[
            pl.BlockSpec((BE, C), lambda i: (i, 0)),
            _full_spec(wh.shape),
            _full_spec(bh.shape),
        ],
        out_specs=pl.BlockSpec((BE, C), lambda i: (i, 0)),
        out_shape=jax.ShapeDtypeStruct((n, C), jnp.float32),
    )(g, wh, bh)


def _mlp5_pair(glo, ghi, ew, wh, bh):
    n = glo.shape[0]
    grid = (n // BE,)
    return pl.pallas_call(
        _mlp5_pair_kernel,
        grid=grid,
        in_specs=[
            pl.BlockSpec((BE, C), lambda i: (i, 0)),
            pl.BlockSpec((BE, C), lambda i: (i, 0)),
            pl.BlockSpec((BE, 1), lambda i: (i, 0)),
            _full_spec(wh.shape),
            _full_spec(bh.shape),
        ],
        out_specs=pl.BlockSpec((BE, C), lambda i: (i, 0)),
        out_shape=jax.ShapeDtypeStruct((n, C), jnp.float32),
    )(glo, ghi, ew, wh, bh)


def _e2_head(glo, ghi, ef1g, eff, ew, wm, bm, we, be, wh, bh):
    n = glo.shape[0]
    grid = (n // BE,)
    return pl.pallas_call(
        _e2_head_kernel,
        grid=grid,
        in_specs=[
            pl.BlockSpec((BE, C), lambda i: (i, 0)),
            pl.BlockSpec((BE, C), lambda i: (i, 0)),
            pl.BlockSpec((BE, C), lambda i: (i, 0)),
            pl.BlockSpec((BE, C), lambda i: (i, 0)),
            pl.BlockSpec((BE, 1), lambda i: (i, 0)),
            _full_spec(wm.shape),
            _full_spec(bm.shape),
            _full_spec(we.shape),
            _full_spec(be.shape),
            _full_spec(wh.shape),
            _full_spec(bh.shape),
        ],
        out_specs=pl.BlockSpec((BE, 2), lambda i: (i, 0)),
        out_shape=jax.ShapeDtypeStruct((n, 2), jnp.float32),
    )(glo, ghi, ef1g, eff, ew, wm, bm, we, be, wh, bh)


def _stack_tail(pars):
    wh = jnp.stack([w for (w, b) in pars[1:]])
    bh = jnp.stack([b for (w, b) in pars[1:]])
    return wh, bh


def kernel(node_features, edge_features_1d, edge_index, angles, edge_weights, params):
    src = edge_index[0].astype(jnp.int32)
    dst = edge_index[1].astype(jnp.int32)
    n_nodes = node_features.shape[0]
    e_und = edge_weights.shape[0]
    src_lo, src_hi = src[:e_und], src[e_und:]
    dst_lo, dst_hi = dst[:e_und], dst[e_und:]

    cnt = jax.ops.segment_sum(jnp.ones_like(dst, jnp.float32), dst, num_segments=n_nodes)
    inv = 1.0 / jnp.maximum(cnt, 1.0)

    def node_stage(x, pars):
        w0, b0 = pars[0]
        ps = x @ w0[:C] + b0[None, :]
        pd = x @ w0[C : 2 * C]
        g = ps[src] + pd[dst] + angles * w0[2 * C][None, :]
        wh, bh = _stack_tail(pars)
        m = _mlp5(g, wh, bh)
        s = jax.ops.segment_sum(m, dst, num_segments=n_nodes)
        return _leaky(s * inv[:, None])

    x1 = node_stage(node_features, params['node_conv1_msg'])

    v0, c0 = params['edge_conv1_msg'][0]
    qs = x1 @ v0[:C] + c0[None, :]
    qd = x1 @ v0[C:]
    glo = qs[src_lo] + qd[dst_lo]
    ghi = qs[src_hi] + qd[dst_hi]
    wh1, bh1 = _stack_tail(params['edge_conv1_msg'])
    ef1 = _leaky(_mlp5_pair(glo, ghi, edge_weights[:, None], wh1, bh1))

    x2 = node_stage(x1, params['node_conv2_msg'])

    u0, d0 = params['edge_conv2_msg'][0]
    rs = x2 @ u0[:C] + d0[None, :]
    rd = x2 @ u0[C:]
    g2lo = rs[src_lo] + rd[dst_lo]
    g2hi = rs[src_hi] + rd[dst_hi]
    wm, bm = _stack_tail(params['edge_conv2_msg'])

    epars = params['edge_conv2_edge']
    we = jnp.stack([epars[0][0][:C]] + [w for (w, b) in epars[1:]])
    be = jnp.stack([b for (w, b) in epars])
    ef1g = ef1 @ epars[0][0][C:]

    (wl1, bl1) = params['out_lcf1'][0]
    (wl2, bl2) = params['out_lcf2'][0]
    eff = edge_features_1d @ wl1[C : C + 16] + edge_weights[:, None] * wl1[C + 16][None, :]
    wh_head = wl1[:C]
    bh_head = jnp.stack([
        bl1,
        wl2[:, 0],
        wl2[:, 1],
        jnp.concatenate([bl2, jnp.zeros((C - 2,), jnp.float32)]),
    ])

    return _e2_head(g2lo, g2hi, ef1g, eff, edge_weights[:, None],
                    wm, bm, we, be, wh_head, bh_head)
